# Initial kernel scaffold; baseline (speedup 1.0000x reference)
import jax
import jax.numpy as jnp
from jax.experimental import pallas as pl


def kernel(op_codes, node_feats, edge_index, edge_values, emb_table, W1, b1, W2, b2, W3, b3):
    raise NotImplementedError("write your pallas kernel here")


# speedup vs baseline: 3.8324x; 3.8324x over previous
import dataclasses

import jax
import jax.numpy as jnp
from jax import lax
from jax.experimental import pallas as pl
from jax.experimental.pallas import tpu as pltpu
from jax.experimental.pallas import tpu_sc as plsc

_N = 10000
_E = 320000
_D = 128
_EMB = 32
_NEMB = 128
_HID = 128

_NC, _NS = 2, 16
_PER_SUB = _E // (_NC * _NS)
_CHUNK = 80
_NCHUNK = _PER_SUB // _CHUNK
_NBLK = _N // _CHUNK

_F32 = jnp.float32


def _dot(a, b):
    return lax.dot_general(a, b, (((1,), (0,)), ((), ())),
                           precision=lax.Precision.HIGHEST,
                           preferred_element_type=_F32)


_BR = 2000


def _proj_kernel(oc_ref, nf_ref, emb_ref, w1_ref, b1_ref, pro_ref):
    ids = oc_ref[...]
    onehot = (ids == lax.broadcasted_iota(jnp.int32, (_BR, _NEMB), 1)).astype(_F32)
    w1 = w1_ref[...]
    t = _dot(emb_ref[...], w1[:_EMB, :])
    pro = _dot(onehot, t) + _dot(nf_ref[...], w1[_EMB:, :]) + b1_ref[...]
    pro_ref[...] = jnp.maximum(pro, 0.0)


def _edge_kernel(pro_hbm, rows_hbm, cols_hbm, vals_hbm, out_hbm,
                 cols_v, rows_v, vals_v, buf, acc):
    c = lax.axis_index("c")
    s = lax.axis_index("s")

    @pl.loop(0, _CHUNK)
    def _(r):
        for d in range(8):
            buf[r, pl.ds(d * 16, 16)] = jnp.zeros((16,), _F32)

    @pl.loop(0, 8)
    def _(k):
        b = k * _NS + s

        @pl.when(b < _NBLK)
        def _():
            pltpu.sync_copy(buf, acc.at[pl.ds(b * _CHUNK, _CHUNK)])

    plsc.subcore_barrier()

    base = (c * _NS + s) * _PER_SUB

    @pl.loop(0, _NCHUNK)
    def _(t):
        e0 = base + t * _CHUNK
        pltpu.sync_copy(cols_hbm.at[pl.ds(e0, _CHUNK)], cols_v)
        pltpu.sync_copy(rows_hbm.at[pl.ds(e0, _CHUNK)], rows_v)
        pltpu.sync_copy(vals_hbm.at[pl.ds(e0, _CHUNK)], vals_v)
        pltpu.sync_copy(pro_hbm.at[cols_v], buf)

        @pl.loop(0, _CHUNK)
        def _(j):
            vv = plsc.load_gather(vals_v, [jnp.full((16,), 0, jnp.int32) + j])
            for d in range(8):
                sl = pl.ds(d * 16, 16)
                buf[j, sl] = buf[j, sl] * vv

        pltpu.sync_copy(buf, acc.at[rows_v], add=True)

    plsc.subcore_barrier()

    @pl.loop(0, 8)
    def _(k):
        b = k * _NS + s

        @pl.when(b < _NBLK)
        def _():
            pltpu.sync_copy(acc.at[pl.ds(b * _CHUNK, _CHUNK)],
                            out_hbm.at[c, pl.ds(b * _CHUNK, _CHUNK)])


def _head_kernel(agg2_ref, oc_ref, nf_ref, emb_ref, w2_ref, b2_ref,
                 w3_ref, b3_ref, out_ref):
    agg = agg2_ref[0] + agg2_ref[1]
    ids = oc_ref[...]
    onehot = (ids == lax.broadcasted_iota(jnp.int32, (_BR, _NEMB), 1)).astype(_F32)
    w2 = w2_ref[...]
    t2 = _dot(emb_ref[...], w2[_D:_D + _EMB, :])
    h = (_dot(agg, w2[:_D, :]) + _dot(onehot, t2)
         + _dot(nf_ref[...], w2[_D + _EMB:, :]) + b2_ref[...])
    h = jnp.maximum(h, 0.0)
    out_ref[...] = _dot(h, w3_ref[...]) + b3_ref[...]


def kernel(op_codes, node_feats, edge_index, edge_values, emb_table,
           W1, b1, W2, b2, W3, b3):
    sc_mesh = plsc.VectorSubcoreMesh(core_axis_name="c", subcore_axis_name="s",
                                     num_cores=_NC, num_subcores=_NS)
    sc_params = pltpu.CompilerParams()
    if "needs_layout_passes" in pltpu.CompilerParams.__dataclass_fields__:
        sc_params = dataclasses.replace(sc_params, needs_layout_passes=False)
    oc2 = op_codes.reshape(_N, 1)

    pro = pl.pallas_call(
        _proj_kernel,
        grid=(_N // _BR,),
        in_specs=[
            pl.BlockSpec((_BR, 1), lambda i: (i, 0)),
            pl.BlockSpec((_BR, 127), lambda i: (i, 0)),
            pl.BlockSpec((_NEMB, _EMB), lambda i: (0, 0)),
            pl.BlockSpec((_EMB + 127, _D), lambda i: (0, 0)),
            pl.BlockSpec((1, _D), lambda i: (0, 0)),
        ],
        out_specs=pl.BlockSpec((_BR, _D), lambda i: (i, 0)),
        out_shape=jax.ShapeDtypeStruct((_N, _D), _F32),
    )(oc2, node_feats, emb_table, W1, b1.reshape(1, _D))

    rows = edge_index[0]
    cols = edge_index[1]
    agg2 = pl.kernel(
        _edge_kernel,
        out_type=jax.ShapeDtypeStruct((_NC, _N, _D), _F32),
        mesh=sc_mesh,
        compiler_params=sc_params,
        scratch_types=[
            pltpu.VMEM((_CHUNK,), jnp.int32),
            pltpu.VMEM((_CHUNK,), jnp.int32),
            pltpu.VMEM((_CHUNK,), _F32),
            pltpu.VMEM((_CHUNK, _D), _F32),
            pltpu.VMEM_SHARED((_N, _D), _F32),
        ],
    )(pro, rows, cols, edge_values)

    out = pl.pallas_call(
        _head_kernel,
        grid=(_N // _BR,),
        in_specs=[
            pl.BlockSpec((_NC, _BR, _D), lambda i: (0, i, 0)),
            pl.BlockSpec((_BR, 1), lambda i: (i, 0)),
            pl.BlockSpec((_BR, 127), lambda i: (i, 0)),
            pl.BlockSpec((_NEMB, _EMB), lambda i: (0, 0)),
            pl.BlockSpec((_D + _EMB + 127, _HID), lambda i: (0, 0)),
            pl.BlockSpec((1, _HID), lambda i: (0, 0)),
            pl.BlockSpec((_HID, 1), lambda i: (0, 0)),
            pl.BlockSpec((1, 1), lambda i: (0, 0)),
        ],
        out_specs=pl.BlockSpec((_BR, 1), lambda i: (i, 0)),
        out_shape=jax.ShapeDtypeStruct((_N, 1), _F32),
    )(agg2, oc2, node_feats, emb_table, W2, b2.reshape(1, _HID), W3,
      b3.reshape(1, 1))
    return out


# speedup vs baseline: 5.3653x; 1.4000x over previous
import dataclasses

import jax
import jax.numpy as jnp
from jax import lax
from jax.experimental import pallas as pl
from jax.experimental.pallas import tpu as pltpu
from jax.experimental.pallas import tpu_sc as plsc

_N = 10000
_E = 320000
_D = 128
_EMB = 32
_NEMB = 128
_HID = 128

_NC, _NS = 2, 16
_PER_SUB = _E // (_NC * _NS)
_CHUNK = 80
_NCHUNK = _PER_SUB // _CHUNK
_NBLK = _N // _CHUNK

_F32 = jnp.float32


def _dot(a, b):
    return lax.dot_general(a, b, (((1,), (0,)), ((), ())),
                           precision=lax.Precision.HIGHEST,
                           preferred_element_type=_F32)


_BR = 2000


def _proj_kernel(oc_ref, nf_ref, emb_ref, w1_ref, b1_ref, pro_ref):
    ids = oc_ref[...]
    onehot = (ids == lax.broadcasted_iota(jnp.int32, (_BR, _NEMB), 1)).astype(_F32)
    w1 = w1_ref[...]
    t = _dot(emb_ref[...], w1[:_EMB, :])
    pro = _dot(onehot, t) + _dot(nf_ref[...], w1[_EMB:, :]) + b1_ref[...]
    pro_ref[...] = jnp.maximum(pro, 0.0)


def _edge_kernel(pro_hbm, rows_hbm, cols_hbm, vals_hbm, out_hbm,
                 cols_a, rows_a, vals_a, buf, acc):
    c = lax.axis_index("c")
    s = lax.axis_index("s")

    base = (c * _NS + s) * _PER_SUB
    pltpu.sync_copy(cols_hbm.at[pl.ds(base, _PER_SUB)], cols_a)
    pltpu.sync_copy(rows_hbm.at[pl.ds(base, _PER_SUB)], rows_a)
    pltpu.sync_copy(vals_hbm.at[pl.ds(base, _PER_SUB)], vals_a)

    @pl.loop(0, _CHUNK)
    def _(r):
        for d in range(8):
            buf[r, pl.ds(d * 16, 16)] = jnp.zeros((16,), _F32)

    @pl.loop(0, 8)
    def _(k):
        b = k * _NS + s

        @pl.when(b < _NBLK)
        def _():
            pltpu.sync_copy(buf, acc.at[pl.ds(b * _CHUNK, _CHUNK)])

    plsc.subcore_barrier()

    @pl.loop(0, _NCHUNK)
    def _(t):
        off = t * _CHUNK
        pltpu.sync_copy(pro_hbm.at[cols_a.at[pl.ds(off, _CHUNK)]], buf)

        @pl.loop(0, _CHUNK)
        def _(j):
            vv = plsc.load_gather(vals_a,
                                  [jnp.full((16,), 0, jnp.int32) + (off + j)])
            for d in range(8):
                sl = pl.ds(d * 16, 16)
                buf[j, sl] = buf[j, sl] * vv

        pltpu.sync_copy(buf, acc.at[rows_a.at[pl.ds(off, _CHUNK)]], add=True)

    plsc.subcore_barrier()

    @pl.loop(0, 8)
    def _(k):
        b = k * _NS + s

        @pl.when(b < _NBLK)
        def _():
            pltpu.sync_copy(acc.at[pl.ds(b * _CHUNK, _CHUNK)],
                            out_hbm.at[c, pl.ds(b * _CHUNK, _CHUNK)])


def _head_kernel(agg2_ref, oc_ref, nf_ref, emb_ref, w2_ref, b2_ref,
                 w3_ref, b3_ref, out_ref):
    agg = agg2_ref[0] + agg2_ref[1]
    ids = oc_ref[...]
    onehot = (ids == lax.broadcasted_iota(jnp.int32, (_BR, _NEMB), 1)).astype(_F32)
    w2 = w2_ref[...]
    t2 = _dot(emb_ref[...], w2[_D:_D + _EMB, :])
    h = (_dot(agg, w2[:_D, :]) + _dot(onehot, t2)
         + _dot(nf_ref[...], w2[_D + _EMB:, :]) + b2_ref[...])
    h = jnp.maximum(h, 0.0)
    out_ref[...] = _dot(h, w3_ref[...]) + b3_ref[...]


def kernel(op_codes, node_feats, edge_index, edge_values, emb_table,
           W1, b1, W2, b2, W3, b3):
    sc_mesh = plsc.VectorSubcoreMesh(core_axis_name="c", subcore_axis_name="s",
                                     num_cores=_NC, num_subcores=_NS)
    sc_params = pltpu.CompilerParams()
    if "needs_layout_passes" in pltpu.CompilerParams.__dataclass_fields__:
        sc_params = dataclasses.replace(sc_params, needs_layout_passes=False)
    oc2 = op_codes.reshape(_N, 1)

    pro = pl.pallas_call(
        _proj_kernel,
        grid=(_N // _BR,),
        in_specs=[
            pl.BlockSpec((_BR, 1), lambda i: (i, 0)),
            pl.BlockSpec((_BR, 127), lambda i: (i, 0)),
            pl.BlockSpec((_NEMB, _EMB), lambda i: (0, 0)),
            pl.BlockSpec((_EMB + 127, _D), lambda i: (0, 0)),
            pl.BlockSpec((1, _D), lambda i: (0, 0)),
        ],
        out_specs=pl.BlockSpec((_BR, _D), lambda i: (i, 0)),
        out_shape=jax.ShapeDtypeStruct((_N, _D), _F32),
    )(oc2, node_feats, emb_table, W1, b1.reshape(1, _D))

    rows = edge_index[0]
    cols = edge_index[1]
    agg2 = pl.kernel(
        _edge_kernel,
        out_type=jax.ShapeDtypeStruct((_NC, _N, _D), _F32),
        mesh=sc_mesh,
        compiler_params=sc_params,
        scratch_types=[
            pltpu.VMEM((_PER_SUB,), jnp.int32),
            pltpu.VMEM((_PER_SUB,), jnp.int32),
            pltpu.VMEM((_PER_SUB,), _F32),
            pltpu.VMEM((_CHUNK, _D), _F32),
            pltpu.VMEM_SHARED((_N, _D), _F32),
        ],
    )(pro, rows, cols, edge_values)

    out = pl.pallas_call(
        _head_kernel,
        grid=(_N // _BR,),
        in_specs=[
            pl.BlockSpec((_NC, _BR, _D), lambda i: (0, i, 0)),
            pl.BlockSpec((_BR, 1), lambda i: (i, 0)),
            pl.BlockSpec((_BR, 127), lambda i: (i, 0)),
            pl.BlockSpec((_NEMB, _EMB), lambda i: (0, 0)),
            pl.BlockSpec((_D + _EMB + 127, _HID), lambda i: (0, 0)),
            pl.BlockSpec((1, _HID), lambda i: (0, 0)),
            pl.BlockSpec((_HID, 1), lambda i: (0, 0)),
            pl.BlockSpec((1, 1), lambda i: (0, 0)),
        ],
        out_specs=pl.BlockSpec((_BR, 1), lambda i: (i, 0)),
        out_shape=jax.ShapeDtypeStruct((_N, 1), _F32),
    )(agg2, oc2, node_feats, emb_table, W2, b2.reshape(1, _HID), W3,
      b3.reshape(1, 1))
    return out


# speedup vs baseline: 8.1489x; 1.5188x over previous
import dataclasses

import jax
import jax.numpy as jnp
from jax import lax
from jax.experimental import pallas as pl
from jax.experimental.pallas import tpu as pltpu
from jax.experimental.pallas import tpu_sc as plsc

_N = 10000
_E = 320000
_D = 128
_EMB = 32
_NEMB = 128
_HID = 128

_NC, _NS = 2, 16
_PER_SUB = _E // (_NC * _NS)
_CHUNK = 80
_NCHUNK = _PER_SUB // _CHUNK
_ZBLK = 80
_NBLK = _N // _ZBLK

_F32 = jnp.float32


def _dot(a, b):
    return lax.dot_general(a, b, (((1,), (0,)), ((), ())),
                           precision=lax.Precision.HIGHEST,
                           preferred_element_type=_F32)


_BR = 2000


def _proj_kernel(oc_ref, nf_ref, emb_ref, w1_ref, b1_ref, pro_ref):
    ids = oc_ref[...]
    onehot = (ids == lax.broadcasted_iota(jnp.int32, (_BR, _NEMB), 1)).astype(_F32)
    w1 = w1_ref[...]
    t = _dot(emb_ref[...], w1[:_EMB, :])
    pro = _dot(onehot, t) + _dot(nf_ref[...], w1[_EMB:, :]) + b1_ref[...]
    pro_ref[...] = jnp.maximum(pro, 0.0)


def _edge_kernel(pro_hbm, rows_hbm, cols_hbm, vals_hbm, out_hbm,
                 cols_a, rows_a, vals_a, buf0, buf1, acc, sem0, sem1):
    c = lax.axis_index("c")
    s = lax.axis_index("s")

    base = (c * _NS + s) * _PER_SUB
    pltpu.sync_copy(cols_hbm.at[pl.ds(base, _PER_SUB)], cols_a)
    pltpu.sync_copy(rows_hbm.at[pl.ds(base, _PER_SUB)], rows_a)
    pltpu.sync_copy(vals_hbm.at[pl.ds(base, _PER_SUB)], vals_a)

    @pl.loop(0, _ZBLK)
    def _(r):
        for d in range(8):
            buf0[r, pl.ds(d * 16, 16)] = jnp.zeros((16,), _F32)

    @pl.loop(0, 8)
    def _(k):
        b = k * _NS + s

        @pl.when(b < _NBLK)
        def _():
            pltpu.sync_copy(buf0.at[pl.ds(0, _ZBLK)], acc.at[pl.ds(b * _ZBLK, _ZBLK)])

    plsc.subcore_barrier()

    def _gather(t, buf, sem):
        idx = cols_a.at[pl.ds(t * _CHUNK, _CHUNK)]
        return pltpu.async_copy(pro_hbm.at[idx], buf, sem)

    def _gather_wait(t, buf, sem):
        idx = cols_a.at[pl.ds(t * _CHUNK, _CHUNK)]
        pltpu.make_async_copy(pro_hbm.at[idx], buf, sem).wait()

    def _process(t, buf, sem, nbuf, nsem):
        _gather_wait(t, buf, sem)

        @pl.when(t + 1 < _NCHUNK)
        def _():
            _gather(t + 1, nbuf, nsem)

        off = t * _CHUNK

        @pl.loop(0, _CHUNK)
        def _(j):
            vv = plsc.load_gather(vals_a,
                                  [jnp.full((16,), 0, jnp.int32) + (off + j)])
            for d in range(8):
                sl = pl.ds(d * 16, 16)
                buf[j, sl] = buf[j, sl] * vv

        pltpu.sync_copy(buf, acc.at[rows_a.at[pl.ds(off, _CHUNK)]], add=True)

    _gather(0, buf0, sem0)

    @pl.loop(0, _NCHUNK // 2)
    def _(u):
        _process(2 * u, buf0, sem0, buf1, sem1)
        _process(2 * u + 1, buf1, sem1, buf0, sem0)

    _process(_NCHUNK - 1, buf0, sem0, buf1, sem1)

    plsc.subcore_barrier()

    @pl.loop(0, 8)
    def _(k):
        b = k * _NS + s

        @pl.when(b < _NBLK)
        def _():
            pltpu.sync_copy(acc.at[pl.ds(b * _ZBLK, _ZBLK)],
                            out_hbm.at[c, pl.ds(b * _ZBLK, _ZBLK)])


def _head_kernel(agg2_ref, oc_ref, nf_ref, emb_ref, w2_ref, b2_ref,
                 w3_ref, b3_ref, out_ref):
    agg = agg2_ref[0] + agg2_ref[1]
    ids = oc_ref[...]
    onehot = (ids == lax.broadcasted_iota(jnp.int32, (_BR, _NEMB), 1)).astype(_F32)
    w2 = w2_ref[...]
    t2 = _dot(emb_ref[...], w2[_D:_D + _EMB, :])
    h = (_dot(agg, w2[:_D, :]) + _dot(onehot, t2)
         + _dot(nf_ref[...], w2[_D + _EMB:, :]) + b2_ref[...])
    h = jnp.maximum(h, 0.0)
    out_ref[...] = _dot(h, w3_ref[...]) + b3_ref[...]


def kernel(op_codes, node_feats, edge_index, edge_values, emb_table,
           W1, b1, W2, b2, W3, b3):
    sc_mesh = plsc.VectorSubcoreMesh(core_axis_name="c", subcore_axis_name="s",
                                     num_cores=_NC, num_subcores=_NS)
    sc_params = pltpu.CompilerParams()
    if "needs_layout_passes" in pltpu.CompilerParams.__dataclass_fields__:
        sc_params = dataclasses.replace(sc_params, needs_layout_passes=False)
    oc2 = op_codes.reshape(_N, 1)

    pro = pl.pallas_call(
        _proj_kernel,
        grid=(_N // _BR,),
        in_specs=[
            pl.BlockSpec((_BR, 1), lambda i: (i, 0)),
            pl.BlockSpec((_BR, 127), lambda i: (i, 0)),
            pl.BlockSpec((_NEMB, _EMB), lambda i: (0, 0)),
            pl.BlockSpec((_EMB + 127, _D), lambda i: (0, 0)),
            pl.BlockSpec((1, _D), lambda i: (0, 0)),
        ],
        out_specs=pl.BlockSpec((_BR, _D), lambda i: (i, 0)),
        out_shape=jax.ShapeDtypeStruct((_N, _D), _F32),
    )(oc2, node_feats, emb_table, W1, b1.reshape(1, _D))

    rows = edge_index[0]
    cols = edge_index[1]
    agg2 = pl.kernel(
        _edge_kernel,
        out_type=jax.ShapeDtypeStruct((_NC, _N, _D), _F32),
        mesh=sc_mesh,
        compiler_params=sc_params,
        scratch_types=[
            pltpu.VMEM((_PER_SUB,), jnp.int32),
            pltpu.VMEM((_PER_SUB,), jnp.int32),
            pltpu.VMEM((_PER_SUB,), _F32),
            pltpu.VMEM((_CHUNK, _D), _F32),
            pltpu.VMEM((_CHUNK, _D), _F32),
            pltpu.VMEM_SHARED((_N, _D), _F32),
            pltpu.SemaphoreType.DMA,
            pltpu.SemaphoreType.DMA,
        ],
    )(pro, rows, cols, edge_values)

    out = pl.pallas_call(
        _head_kernel,
        grid=(_N // _BR,),
        in_specs=[
            pl.BlockSpec((_NC, _BR, _D), lambda i: (0, i, 0)),
            pl.BlockSpec((_BR, 1), lambda i: (i, 0)),
            pl.BlockSpec((_BR, 127), lambda i: (i, 0)),
            pl.BlockSpec((_NEMB, _EMB), lambda i: (0, 0)),
            pl.BlockSpec((_D + _EMB + 127, _HID), lambda i: (0, 0)),
            pl.BlockSpec((1, _HID), lambda i: (0, 0)),
            pl.BlockSpec((_HID, 1), lambda i: (0, 0)),
            pl.BlockSpec((1, 1), lambda i: (0, 0)),
        ],
        out_specs=pl.BlockSpec((_BR, 1), lambda i: (i, 0)),
        out_shape=jax.ShapeDtypeStruct((_N, 1), _F32),
    )(agg2, oc2, node_feats, emb_table, W2, b2.reshape(1, _HID), W3,
      b3.reshape(1, 1))
    return out


# speedup vs baseline: 8.7663x; 1.0758x over previous
import dataclasses

import jax
import jax.numpy as jnp
from jax import lax
from jax.experimental import pallas as pl
from jax.experimental.pallas import tpu as pltpu
from jax.experimental.pallas import tpu_sc as plsc

_N = 10000
_E = 320000
_D = 128
_EMB = 32
_NEMB = 128
_HID = 128

_NC, _NS = 2, 16
_PER_SUB = _E // (_NC * _NS)
_CHUNK = 80
_NCHUNK = _PER_SUB // _CHUNK
_ZBLK = 80
_NBLK = _N // _ZBLK

_F32 = jnp.float32


def _dot(a, b):
    return lax.dot_general(a, b, (((1,), (0,)), ((), ())),
                           precision=lax.Precision.HIGHEST,
                           preferred_element_type=_F32)


_BR = 2000


def _proj_kernel(oc_ref, nf_ref, emb_ref, w1_ref, b1_ref, pro_ref):
    ids = oc_ref[...]
    onehot = (ids == lax.broadcasted_iota(jnp.int32, (_BR, _NEMB), 1)).astype(_F32)
    w1 = w1_ref[...]
    t = _dot(emb_ref[...], w1[:_EMB, :])
    pro = _dot(onehot, t) + _dot(nf_ref[...], w1[_EMB:, :]) + b1_ref[...]
    pro_ref[...] = jnp.maximum(pro, 0.0)


def _edge_kernel(pro_hbm, rows_hbm, cols_hbm, vals_hbm, out_hbm,
                 cols_a, rows_a, vals_a, buf0, buf1, acc, sem0, sem1,
                 ssem0, ssem1):
    c = lax.axis_index("c")
    s = lax.axis_index("s")

    base = (c * _NS + s) * _PER_SUB
    pltpu.sync_copy(cols_hbm.at[pl.ds(base, _PER_SUB)], cols_a)
    pltpu.sync_copy(rows_hbm.at[pl.ds(base, _PER_SUB)], rows_a)
    pltpu.sync_copy(vals_hbm.at[pl.ds(base, _PER_SUB)], vals_a)

    @pl.loop(0, _ZBLK)
    def _(r):
        for d in range(8):
            buf0[r, pl.ds(d * 16, 16)] = jnp.zeros((16,), _F32)

    @pl.loop(0, 8)
    def _(k):
        b = k * _NS + s

        @pl.when(b < _NBLK)
        def _():
            pltpu.sync_copy(buf0.at[pl.ds(0, _ZBLK)], acc.at[pl.ds(b * _ZBLK, _ZBLK)])

    plsc.subcore_barrier()

    def _gather(t, buf, sem):
        idx = cols_a.at[pl.ds(t * _CHUNK, _CHUNK)]
        return pltpu.async_copy(pro_hbm.at[idx], buf, sem)

    def _gather_wait(t, buf, sem):
        idx = cols_a.at[pl.ds(t * _CHUNK, _CHUNK)]
        pltpu.make_async_copy(pro_hbm.at[idx], buf, sem).wait()

    def _scatter(t, buf, sem):
        idx = rows_a.at[pl.ds(t * _CHUNK, _CHUNK)]
        pltpu.async_copy(buf, acc.at[idx], sem, add=True)

    def _scatter_wait(t, buf, sem):
        idx = rows_a.at[pl.ds(t * _CHUNK, _CHUNK)]
        pltpu.make_async_copy(buf, acc.at[idx], sem).wait()

    def _process(t, buf, sem, ssem, nbuf, nsem, nssem):
        _gather_wait(t, buf, sem)

        @pl.when(t + 1 < _NCHUNK)
        def _():
            @pl.when(t >= 1)
            def _():
                _scatter_wait(t - 1, nbuf, nssem)

            _gather(t + 1, nbuf, nsem)

        off = t * _CHUNK

        @plsc.parallel_loop(0, _CHUNK, unroll=4)
        def _(j):
            vv = plsc.load_gather(vals_a,
                                  [jnp.full((16,), 0, jnp.int32) + (off + j)])
            for d in range(8):
                sl = pl.ds(d * 16, 16)
                buf[j, sl] = buf[j, sl] * vv

        _scatter(t, buf, ssem)

    _gather(0, buf0, sem0)

    @pl.loop(0, _NCHUNK // 2)
    def _(u):
        _process(2 * u, buf0, sem0, ssem0, buf1, sem1, ssem1)
        _process(2 * u + 1, buf1, sem1, ssem1, buf0, sem0, ssem0)

    _process(_NCHUNK - 1, buf0, sem0, ssem0, buf1, sem1, ssem1)

    _scatter_wait(_NCHUNK - 2, buf1, ssem1)
    _scatter_wait(_NCHUNK - 1, buf0, ssem0)

    plsc.subcore_barrier()

    @pl.loop(0, 8)
    def _(k):
        b = k * _NS + s

        @pl.when(b < _NBLK)
        def _():
            pltpu.sync_copy(acc.at[pl.ds(b * _ZBLK, _ZBLK)],
                            out_hbm.at[c, pl.ds(b * _ZBLK, _ZBLK)])


def _head_kernel(agg2_ref, oc_ref, nf_ref, emb_ref, w2_ref, b2_ref,
                 w3_ref, b3_ref, out_ref):
    agg = agg2_ref[0] + agg2_ref[1]
    ids = oc_ref[...]
    onehot = (ids == lax.broadcasted_iota(jnp.int32, (_BR, _NEMB), 1)).astype(_F32)
    w2 = w2_ref[...]
    t2 = _dot(emb_ref[...], w2[_D:_D + _EMB, :])
    h = (_dot(agg, w2[:_D, :]) + _dot(onehot, t2)
         + _dot(nf_ref[...], w2[_D + _EMB:, :]) + b2_ref[...])
    h = jnp.maximum(h, 0.0)
    out_ref[...] = _dot(h, w3_ref[...]) + b3_ref[...]


def kernel(op_codes, node_feats, edge_index, edge_values, emb_table,
           W1, b1, W2, b2, W3, b3):
    sc_mesh = plsc.VectorSubcoreMesh(core_axis_name="c", subcore_axis_name="s",
                                     num_cores=_NC, num_subcores=_NS)
    sc_params = pltpu.CompilerParams()
    if "needs_layout_passes" in pltpu.CompilerParams.__dataclass_fields__:
        sc_params = dataclasses.replace(sc_params, needs_layout_passes=False)
    oc2 = op_codes.reshape(_N, 1)

    pro = pl.pallas_call(
        _proj_kernel,
        grid=(_N // _BR,),
        in_specs=[
            pl.BlockSpec((_BR, 1), lambda i: (i, 0)),
            pl.BlockSpec((_BR, 127), lambda i: (i, 0)),
            pl.BlockSpec((_NEMB, _EMB), lambda i: (0, 0)),
            pl.BlockSpec((_EMB + 127, _D), lambda i: (0, 0)),
            pl.BlockSpec((1, _D), lambda i: (0, 0)),
        ],
        out_specs=pl.BlockSpec((_BR, _D), lambda i: (i, 0)),
        out_shape=jax.ShapeDtypeStruct((_N, _D), _F32),
    )(oc2, node_feats, emb_table, W1, b1.reshape(1, _D))

    rows = edge_index[0]
    cols = edge_index[1]
    agg2 = pl.kernel(
        _edge_kernel,
        out_type=jax.ShapeDtypeStruct((_NC, _N, _D), _F32),
        mesh=sc_mesh,
        compiler_params=sc_params,
        scratch_types=[
            pltpu.VMEM((_PER_SUB,), jnp.int32),
            pltpu.VMEM((_PER_SUB,), jnp.int32),
            pltpu.VMEM((_PER_SUB,), _F32),
            pltpu.VMEM((_CHUNK, _D), _F32),
            pltpu.VMEM((_CHUNK, _D), _F32),
            pltpu.VMEM_SHARED((_N, _D), _F32),
            pltpu.SemaphoreType.DMA,
            pltpu.SemaphoreType.DMA,
            pltpu.SemaphoreType.DMA,
            pltpu.SemaphoreType.DMA,
        ],
    )(pro, rows, cols, edge_values)

    out = pl.pallas_call(
        _head_kernel,
        grid=(_N // _BR,),
        in_specs=[
            pl.BlockSpec((_NC, _BR, _D), lambda i: (0, i, 0)),
            pl.BlockSpec((_BR, 1), lambda i: (i, 0)),
            pl.BlockSpec((_BR, 127), lambda i: (i, 0)),
            pl.BlockSpec((_NEMB, _EMB), lambda i: (0, 0)),
            pl.BlockSpec((_D + _EMB + 127, _HID), lambda i: (0, 0)),
            pl.BlockSpec((1, _HID), lambda i: (0, 0)),
            pl.BlockSpec((_HID, 1), lambda i: (0, 0)),
            pl.BlockSpec((1, 1), lambda i: (0, 0)),
        ],
        out_specs=pl.BlockSpec((_BR, 1), lambda i: (i, 0)),
        out_shape=jax.ShapeDtypeStruct((_N, 1), _F32),
    )(agg2, oc2, node_feats, emb_table, W2, b2.reshape(1, _HID), W3,
      b3.reshape(1, 1))
    return out


# speedup vs baseline: 9.5877x; 1.0937x over previous
import dataclasses

import jax
import jax.numpy as jnp
from jax import lax
from jax.experimental import pallas as pl
from jax.experimental.pallas import tpu as pltpu
from jax.experimental.pallas import tpu_sc as plsc

_N = 10000
_E = 320000
_D = 128
_EMB = 32
_NEMB = 128
_HID = 128

_NC, _NS = 2, 16
_PER_SUB = _E // (_NC * _NS)
_CHUNK = 40
_NCHUNK = _PER_SUB // _CHUNK
_NBUF = 4
_ZBLK = 40
_NBLK = _N // _ZBLK

_F32 = jnp.float32


def _dot(a, b):
    return lax.dot_general(a, b, (((1,), (0,)), ((), ())),
                           precision=lax.Precision.HIGHEST,
                           preferred_element_type=_F32)


_BR = 2000


def _proj_kernel(oc_ref, nf_ref, emb_ref, w1_ref, b1_ref, pro_ref):
    ids = oc_ref[...]
    onehot = (ids == lax.broadcasted_iota(jnp.int32, (_BR, _NEMB), 1)).astype(_F32)
    w1 = w1_ref[...]
    t = _dot(emb_ref[...], w1[:_EMB, :])
    pro = _dot(onehot, t) + _dot(nf_ref[...], w1[_EMB:, :]) + b1_ref[...]
    pro_ref[...] = jnp.maximum(pro, 0.0)


def _edge_kernel(pro_hbm, rows_hbm, cols_hbm, vals_hbm, out_hbm,
                 cols_a, rows_a, vals_a, b0, b1, b2, b3, acc,
                 g0, g1, g2, g3, s0, s1, s2, s3):
    bufs = [b0, b1, b2, b3]
    gsems = [g0, g1, g2, g3]
    ssems = [s0, s1, s2, s3]
    c = lax.axis_index("c")
    s = lax.axis_index("s")

    base = (c * _NS + s) * _PER_SUB
    pltpu.sync_copy(cols_hbm.at[pl.ds(base, _PER_SUB)], cols_a)
    pltpu.sync_copy(rows_hbm.at[pl.ds(base, _PER_SUB)], rows_a)
    pltpu.sync_copy(vals_hbm.at[pl.ds(base, _PER_SUB)], vals_a)

    @pl.loop(0, _ZBLK)
    def _(r):
        for d in range(8):
            b0[r, pl.ds(d * 16, 16)] = jnp.zeros((16,), _F32)

    @pl.loop(0, 16)
    def _(k):
        b = k * _NS + s

        @pl.when(b < _NBLK)
        def _():
            pltpu.sync_copy(b0, acc.at[pl.ds(b * _ZBLK, _ZBLK)])

    plsc.subcore_barrier()

    def _gather(t, buf, sem):
        idx = cols_a.at[pl.ds(t * _CHUNK, _CHUNK)]
        return pltpu.async_copy(pro_hbm.at[idx], buf, sem)

    def _gather_wait(t, buf, sem):
        idx = cols_a.at[pl.ds(t * _CHUNK, _CHUNK)]
        pltpu.make_async_copy(pro_hbm.at[idx], buf, sem).wait()

    def _scatter(t, buf, sem):
        idx = rows_a.at[pl.ds(t * _CHUNK, _CHUNK)]
        pltpu.async_copy(buf, acc.at[idx], sem, add=True)

    def _scatter_wait(t, buf, sem):
        idx = rows_a.at[pl.ds(t * _CHUNK, _CHUNK)]
        pltpu.make_async_copy(buf, acc.at[idx], sem).wait()

    def _process(t, buf, gsem, ssem, pbuf, pgsem, pssem):
        _gather_wait(t, buf, gsem)

        @pl.when(t + _NBUF - 1 < _NCHUNK)
        def _():
            @pl.when(t >= 1)
            def _():
                _scatter_wait(t - 1, pbuf, pssem)

            _gather(t + _NBUF - 1, pbuf, pgsem)

        off = t * _CHUNK

        @plsc.parallel_loop(0, _CHUNK, unroll=4)
        def _(j):
            vv = plsc.load_gather(vals_a,
                                  [jnp.full((16,), 0, jnp.int32) + (off + j)])
            for d in range(8):
                sl = pl.ds(d * 16, 16)
                buf[j, sl] = buf[j, sl] * vv

        _scatter(t, buf, ssem)

    for r in range(_NBUF - 1):
        _gather(r, bufs[r], gsems[r])

    @pl.loop(0, _NCHUNK // _NBUF)
    def _(u):
        t0 = u * _NBUF
        for r in range(_NBUF):
            p = (r + _NBUF - 1) % _NBUF
            _process(t0 + r, bufs[r], gsems[r], ssems[r],
                     bufs[p], gsems[p], ssems[p])

    for t in range(_NCHUNK - _NCHUNK % _NBUF, _NCHUNK):
        r = t % _NBUF
        p = (r + _NBUF - 1) % _NBUF
        _process(t, bufs[r], gsems[r], ssems[r], bufs[p], gsems[p], ssems[p])

    for t in range(_NCHUNK - _NBUF, _NCHUNK):
        _scatter_wait(t, bufs[t % _NBUF], ssems[t % _NBUF])

    plsc.subcore_barrier()

    @pl.loop(0, 16)
    def _(k):
        b = k * _NS + s

        @pl.when(b < _NBLK)
        def _():
            pltpu.sync_copy(acc.at[pl.ds(b * _ZBLK, _ZBLK)],
                            out_hbm.at[c, pl.ds(b * _ZBLK, _ZBLK)])


def _head_kernel(agg2_ref, oc_ref, nf_ref, emb_ref, w2_ref, b2_ref,
                 w3_ref, b3_ref, out_ref):
    agg = agg2_ref[0] + agg2_ref[1]
    ids = oc_ref[...]
    onehot = (ids == lax.broadcasted_iota(jnp.int32, (_BR, _NEMB), 1)).astype(_F32)
    w2 = w2_ref[...]
    t2 = _dot(emb_ref[...], w2[_D:_D + _EMB, :])
    h = (_dot(agg, w2[:_D, :]) + _dot(onehot, t2)
         + _dot(nf_ref[...], w2[_D + _EMB:, :]) + b2_ref[...])
    h = jnp.maximum(h, 0.0)
    out_ref[...] = _dot(h, w3_ref[...]) + b3_ref[...]


def kernel(op_codes, node_feats, edge_index, edge_values, emb_table,
           W1, b1, W2, b2, W3, b3):
    sc_mesh = plsc.VectorSubcoreMesh(core_axis_name="c", subcore_axis_name="s",
                                     num_cores=_NC, num_subcores=_NS)
    sc_params = pltpu.CompilerParams()
    if "needs_layout_passes" in pltpu.CompilerParams.__dataclass_fields__:
        sc_params = dataclasses.replace(sc_params, needs_layout_passes=False)
    oc2 = op_codes.reshape(_N, 1)

    pro = pl.pallas_call(
        _proj_kernel,
        grid=(_N // _BR,),
        in_specs=[
            pl.BlockSpec((_BR, 1), lambda i: (i, 0)),
            pl.BlockSpec((_BR, 127), lambda i: (i, 0)),
            pl.BlockSpec((_NEMB, _EMB), lambda i: (0, 0)),
            pl.BlockSpec((_EMB + 127, _D), lambda i: (0, 0)),
            pl.BlockSpec((1, _D), lambda i: (0, 0)),
        ],
        out_specs=pl.BlockSpec((_BR, _D), lambda i: (i, 0)),
        out_shape=jax.ShapeDtypeStruct((_N, _D), _F32),
    )(oc2, node_feats, emb_table, W1, b1.reshape(1, _D))

    rows = edge_index[0]
    cols = edge_index[1]
    agg2 = pl.kernel(
        _edge_kernel,
        out_type=jax.ShapeDtypeStruct((_NC, _N, _D), _F32),
        mesh=sc_mesh,
        compiler_params=sc_params,
        scratch_types=[
            pltpu.VMEM((_PER_SUB,), jnp.int32),
            pltpu.VMEM((_PER_SUB,), jnp.int32),
            pltpu.VMEM((_PER_SUB,), _F32),
            pltpu.VMEM((_CHUNK, _D), _F32),
            pltpu.VMEM((_CHUNK, _D), _F32),
            pltpu.VMEM((_CHUNK, _D), _F32),
            pltpu.VMEM((_CHUNK, _D), _F32),
            pltpu.VMEM_SHARED((_N, _D), _F32),
            pltpu.SemaphoreType.DMA,
            pltpu.SemaphoreType.DMA,
            pltpu.SemaphoreType.DMA,
            pltpu.SemaphoreType.DMA,
            pltpu.SemaphoreType.DMA,
            pltpu.SemaphoreType.DMA,
            pltpu.SemaphoreType.DMA,
            pltpu.SemaphoreType.DMA,
        ],
    )(pro, rows, cols, edge_values)

    out = pl.pallas_call(
        _head_kernel,
        grid=(_N // _BR,),
        in_specs=[
            pl.BlockSpec((_NC, _BR, _D), lambda i: (0, i, 0)),
            pl.BlockSpec((_BR, 1), lambda i: (i, 0)),
            pl.BlockSpec((_BR, 127), lambda i: (i, 0)),
            pl.BlockSpec((_NEMB, _EMB), lambda i: (0, 0)),
            pl.BlockSpec((_D + _EMB + 127, _HID), lambda i: (0, 0)),
            pl.BlockSpec((1, _HID), lambda i: (0, 0)),
            pl.BlockSpec((_HID, 1), lambda i: (0, 0)),
            pl.BlockSpec((1, 1), lambda i: (0, 0)),
        ],
        out_specs=pl.BlockSpec((_BR, 1), lambda i: (i, 0)),
        out_shape=jax.ShapeDtypeStruct((_N, 1), _F32),
    )(agg2, oc2, node_feats, emb_table, W2, b2.reshape(1, _HID), W3,
      b3.reshape(1, 1))
    return out


# speedup vs baseline: 9.7580x; 1.0178x over previous
import dataclasses

import jax
import jax.numpy as jnp
from jax import lax
from jax.experimental import pallas as pl
from jax.experimental.pallas import tpu as pltpu
from jax.experimental.pallas import tpu_sc as plsc

_N = 10000
_E = 320000
_D = 128
_EMB = 32
_NEMB = 128
_HID = 128

_NC, _NS = 2, 16
_PER_SUB = _E // (_NC * _NS)
_CHUNK = 40
_NCHUNK = _PER_SUB // _CHUNK
_NBUF = 4
_ZBLK = 40
_NBLK = _N // _ZBLK

_F32 = jnp.float32


def _dot(a, b):
    return lax.dot_general(a, b, (((1,), (0,)), ((), ())),
                           precision=lax.Precision.HIGHEST,
                           preferred_element_type=_F32)


_BR = 2000


def _proj_kernel(oc_ref, nf_ref, emb_ref, w1_ref, b1_ref, pro_ref):
    ids = oc_ref[...]
    onehot = (ids == lax.broadcasted_iota(jnp.int32, (_BR, _NEMB), 1)).astype(_F32)
    w1 = w1_ref[...]
    t = _dot(emb_ref[...], w1[:_EMB, :])
    pro = _dot(onehot, t) + _dot(nf_ref[...], w1[_EMB:, :]) + b1_ref[...]
    pro_ref[...] = jnp.maximum(pro, 0.0)


def _edge_kernel(pro_hbm, rows_hbm, cols_hbm, vals_hbm, out_hbm,
                 cols_a, rows_a, vals_a, b0, b1, b2, b3, acc,
                 g0, g1, g2, g3, s0, s1, s2, s3):
    bufs = [b0, b1, b2, b3]
    gsems = [g0, g1, g2, g3]
    ssems = [s0, s1, s2, s3]
    c = lax.axis_index("c")
    s = lax.axis_index("s")

    base = (c * _NS + s) * _PER_SUB
    pltpu.async_copy(cols_hbm.at[pl.ds(base, _PER_SUB)], cols_a, g0)
    pltpu.async_copy(rows_hbm.at[pl.ds(base, _PER_SUB)], rows_a, g1)
    pltpu.async_copy(vals_hbm.at[pl.ds(base, _PER_SUB)], vals_a, g2)

    @pl.loop(0, _ZBLK)
    def _(r):
        for d in range(8):
            b0[r, pl.ds(d * 16, 16)] = jnp.zeros((16,), _F32)

    @pl.loop(0, 16)
    def _(k):
        b = k * _NS + s

        @pl.when(b < _NBLK)
        def _():
            pltpu.async_copy(b0, acc.at[pl.ds(b * _ZBLK, _ZBLK)], g3)

    @pl.loop(0, 16)
    def _(k):
        b = k * _NS + s

        @pl.when(b < _NBLK)
        def _():
            pltpu.make_async_copy(b0, acc.at[pl.ds(b * _ZBLK, _ZBLK)], g3).wait()

    pltpu.make_async_copy(cols_hbm.at[pl.ds(base, _PER_SUB)], cols_a, g0).wait()
    pltpu.make_async_copy(rows_hbm.at[pl.ds(base, _PER_SUB)], rows_a, g1).wait()
    pltpu.make_async_copy(vals_hbm.at[pl.ds(base, _PER_SUB)], vals_a, g2).wait()

    plsc.subcore_barrier()

    def _gather(t, buf, sem):
        idx = cols_a.at[pl.ds(t * _CHUNK, _CHUNK)]
        return pltpu.async_copy(pro_hbm.at[idx], buf, sem)

    def _gather_wait(t, buf, sem):
        idx = cols_a.at[pl.ds(t * _CHUNK, _CHUNK)]
        pltpu.make_async_copy(pro_hbm.at[idx], buf, sem).wait()

    def _scatter(t, buf, sem):
        idx = rows_a.at[pl.ds(t * _CHUNK, _CHUNK)]
        pltpu.async_copy(buf, acc.at[idx], sem, add=True)

    def _scatter_wait(t, buf, sem):
        idx = rows_a.at[pl.ds(t * _CHUNK, _CHUNK)]
        pltpu.make_async_copy(buf, acc.at[idx], sem).wait()

    def _process(t, buf, gsem, ssem, pbuf, pgsem, pssem):
        _gather_wait(t, buf, gsem)

        @pl.when(t + _NBUF - 1 < _NCHUNK)
        def _():
            @pl.when(t >= 1)
            def _():
                _scatter_wait(t - 1, pbuf, pssem)

            _gather(t + _NBUF - 1, pbuf, pgsem)

        off = t * _CHUNK

        @plsc.parallel_loop(0, _CHUNK, unroll=4)
        def _(j):
            vv = plsc.load_gather(vals_a,
                                  [jnp.full((16,), 0, jnp.int32) + (off + j)])
            for d in range(8):
                sl = pl.ds(d * 16, 16)
                buf[j, sl] = buf[j, sl] * vv

        _scatter(t, buf, ssem)

    for r in range(_NBUF - 1):
        _gather(r, bufs[r], gsems[r])

    @pl.loop(0, _NCHUNK // _NBUF)
    def _(u):
        t0 = u * _NBUF
        for r in range(_NBUF):
            p = (r + _NBUF - 1) % _NBUF
            _process(t0 + r, bufs[r], gsems[r], ssems[r],
                     bufs[p], gsems[p], ssems[p])

    for t in range(_NCHUNK - _NCHUNK % _NBUF, _NCHUNK):
        r = t % _NBUF
        p = (r + _NBUF - 1) % _NBUF
        _process(t, bufs[r], gsems[r], ssems[r], bufs[p], gsems[p], ssems[p])

    for t in range(_NCHUNK - _NBUF, _NCHUNK):
        _scatter_wait(t, bufs[t % _NBUF], ssems[t % _NBUF])

    plsc.subcore_barrier()

    @pl.loop(0, 16)
    def _(k):
        b = k * _NS + s

        @pl.when(b < _NBLK)
        def _():
            pltpu.sync_copy(acc.at[pl.ds(b * _ZBLK, _ZBLK)],
                            out_hbm.at[c, pl.ds(b * _ZBLK, _ZBLK)])


def _head_kernel(agg2_ref, oc_ref, nf_ref, emb_ref, w2_ref, b2_ref,
                 w3_ref, b3_ref, out_ref):
    agg = agg2_ref[0] + agg2_ref[1]
    ids = oc_ref[...]
    onehot = (ids == lax.broadcasted_iota(jnp.int32, (_BR, _NEMB), 1)).astype(_F32)
    w2 = w2_ref[...]
    t2 = _dot(emb_ref[...], w2[_D:_D + _EMB, :])
    h = (_dot(agg, w2[:_D, :]) + _dot(onehot, t2)
         + _dot(nf_ref[...], w2[_D + _EMB:, :]) + b2_ref[...])
    h = jnp.maximum(h, 0.0)
    out_ref[...] = _dot(h, w3_ref[...]) + b3_ref[...]


def kernel(op_codes, node_feats, edge_index, edge_values, emb_table,
           W1, b1, W2, b2, W3, b3):
    sc_mesh = plsc.VectorSubcoreMesh(core_axis_name="c", subcore_axis_name="s",
                                     num_cores=_NC, num_subcores=_NS)
    sc_params = pltpu.CompilerParams()
    if "needs_layout_passes" in pltpu.CompilerParams.__dataclass_fields__:
        sc_params = dataclasses.replace(sc_params, needs_layout_passes=False)
    oc2 = op_codes.reshape(_N, 1)

    pro = pl.pallas_call(
        _proj_kernel,
        grid=(_N // _BR,),
        in_specs=[
            pl.BlockSpec((_BR, 1), lambda i: (i, 0)),
            pl.BlockSpec((_BR, 127), lambda i: (i, 0)),
            pl.BlockSpec((_NEMB, _EMB), lambda i: (0, 0)),
            pl.BlockSpec((_EMB + 127, _D), lambda i: (0, 0)),
            pl.BlockSpec((1, _D), lambda i: (0, 0)),
        ],
        out_specs=pl.BlockSpec((_BR, _D), lambda i: (i, 0)),
        out_shape=jax.ShapeDtypeStruct((_N, _D), _F32),
    )(oc2, node_feats, emb_table, W1, b1.reshape(1, _D))

    rows = edge_index[0]
    cols = edge_index[1]
    agg2 = pl.kernel(
        _edge_kernel,
        out_type=jax.ShapeDtypeStruct((_NC, _N, _D), _F32),
        mesh=sc_mesh,
        compiler_params=sc_params,
        scratch_types=[
            pltpu.VMEM((_PER_SUB,), jnp.int32),
            pltpu.VMEM((_PER_SUB,), jnp.int32),
            pltpu.VMEM((_PER_SUB,), _F32),
            pltpu.VMEM((_CHUNK, _D), _F32),
            pltpu.VMEM((_CHUNK, _D), _F32),
            pltpu.VMEM((_CHUNK, _D), _F32),
            pltpu.VMEM((_CHUNK, _D), _F32),
            pltpu.VMEM_SHARED((_N, _D), _F32),
            pltpu.SemaphoreType.DMA,
            pltpu.SemaphoreType.DMA,
            pltpu.SemaphoreType.DMA,
            pltpu.SemaphoreType.DMA,
            pltpu.SemaphoreType.DMA,
            pltpu.SemaphoreType.DMA,
            pltpu.SemaphoreType.DMA,
            pltpu.SemaphoreType.DMA,
        ],
    )(pro, rows, cols, edge_values)

    out = pl.pallas_call(
        _head_kernel,
        grid=(_N // _BR,),
        in_specs=[
            pl.BlockSpec((_NC, _BR, _D), lambda i: (0, i, 0)),
            pl.BlockSpec((_BR, 1), lambda i: (i, 0)),
            pl.BlockSpec((_BR, 127), lambda i: (i, 0)),
            pl.BlockSpec((_NEMB, _EMB), lambda i: (0, 0)),
            pl.BlockSpec((_D + _EMB + 127, _HID), lambda i: (0, 0)),
            pl.BlockSpec((1, _HID), lambda i: (0, 0)),
            pl.BlockSpec((_HID, 1), lambda i: (0, 0)),
            pl.BlockSpec((1, 1), lambda i: (0, 0)),
        ],
        out_specs=pl.BlockSpec((_BR, 1), lambda i: (i, 0)),
        out_shape=jax.ShapeDtypeStruct((_N, 1), _F32),
    )(agg2, oc2, node_feats, emb_table, W2, b2.reshape(1, _HID), W3,
      b3.reshape(1, 1))
    return out
